# baseline (speedup 1.0000x reference)
import functools

import jax
import jax.numpy as jnp
from jax import lax
from jax.experimental import pallas as pl
from jax.experimental.pallas import tpu as pltpu
from jax.experimental.pallas import tpu_sc as plsc

NUM_EMB = 8192
DIM = 64
ROWS = 16384
COMMITMENT_COST = 0.25

BLK_R = 256
BLK_K = 256
N_R = ROWS // BLK_R
N_KC = NUM_EMB // BLK_K
HW = 1024
R_PER_IMG = HW // BLK_R


def _dist_body(x_ref, rn_ref, w_ref, wn_ref, idx_ref, md_ref):
    row = pl.program_id(1)
    x_t = x_ref[...].reshape(DIM, BLK_R)
    rn = rn_ref[...]
    fiota0 = lax.broadcasted_iota(
        jnp.int32, (BLK_K, BLK_R), 0).astype(jnp.float32)
    s = None
    idx = None
    for c in range(N_KC):
        w = w_ref[pl.ds(c * BLK_K, BLK_K), :]
        wn = wn_ref[pl.ds(c * BLK_K, BLK_K), :]
        mm = lax.dot_general(w, x_t, (((1,), (0,)), ((), ())),
                             preferred_element_type=jnp.float32)
        d2 = (rn + wn) - 2.0 * mm
        dist = jnp.sqrt(jnp.maximum(d2, 0.0))
        s_c = jnp.min(dist, axis=0, keepdims=True)
        i_c = jnp.min(jnp.where(dist == s_c, fiota0, 65536.0), axis=0,
                      keepdims=True) + jnp.float32(c * BLK_K)
        if c == 0:
            s, idx = s_c, i_c
        else:
            idx = jnp.where(s_c < s, i_c, idx)
            s = jnp.minimum(s, s_c)
    idx_ref[pl.ds(row, 1), :] = idx.astype(jnp.int32)
    md_ref[pl.ds(row, 1), :] = s * s


def _assign_codes(x3, rn, weight, wn, interpret=False):
    return pl.pallas_call(
        _dist_body,
        grid=(N_R // 8, 8),
        in_specs=[
            pl.BlockSpec((1, DIM, BLK_R),
                         lambda i, j: ((8 * i + j) // R_PER_IMG, 0,
                                       (8 * i + j) % R_PER_IMG)),
            pl.BlockSpec((1, BLK_R), lambda i, j: (0, 8 * i + j)),
            pl.BlockSpec((NUM_EMB, DIM), lambda i, j: (0, 0)),
            pl.BlockSpec((NUM_EMB, 1), lambda i, j: (0, 0)),
        ],
        out_specs=[
            pl.BlockSpec((8, BLK_R), lambda i, j: (i, 0)),
            pl.BlockSpec((8, BLK_R), lambda i, j: (i, 0)),
        ],
        out_shape=[
            jax.ShapeDtypeStruct((N_R, BLK_R), jnp.int32),
            jax.ShapeDtypeStruct((N_R, BLK_R), jnp.float32),
        ],
        compiler_params=pltpu.CompilerParams(
            dimension_semantics=("parallel", "arbitrary")),
        interpret=interpret,
    )(x3, rn, weight, wn)


_NW = 32
_B_PER_W = ROWS // _NW
_IDX_CH = 128
_CH_PER_W = _B_PER_W // _IDX_CH
_GDIM = 128


def _make_sc_gather():
    mesh = plsc.VectorSubcoreMesh(core_axis_name="c", subcore_axis_name="s")

    @functools.partial(
        pl.kernel,
        mesh=mesh,
        out_type=jax.ShapeDtypeStruct((ROWS, _GDIM), jnp.float32),
        scratch_types=[
            pltpu.VMEM((_CH_PER_W, _IDX_CH), jnp.int32),
            pltpu.VMEM((_B_PER_W, _GDIM), jnp.float32),
            pltpu.SemaphoreType.DMA,
        ],
    )
    def gather_k(table_hbm, idx_hbm, out_hbm, idx_v, rows_v, sem):
        wid = lax.axis_index("s") * 2 + lax.axis_index("c")
        base = wid * _B_PER_W
        pltpu.sync_copy(idx_hbm.at[pl.ds(wid * _CH_PER_W, _CH_PER_W)], idx_v)
        copies = []
        for j in range(_CH_PER_W):
            copies.append(pltpu.async_copy(
                table_hbm.at[idx_v.at[j]],
                rows_v.at[pl.ds(j * _IDX_CH, _IDX_CH)], sem))
        for cp in copies:
            cp.wait()
        pltpu.sync_copy(rows_v, out_hbm.at[pl.ds(base, _B_PER_W)])

    return gather_k


_sc_gather_cache = []


def _sc_gather(table, idx2d):
    if not _sc_gather_cache:
        _sc_gather_cache.append(_make_sc_gather())
    return _sc_gather_cache[0](table, idx2d)


def kernel(inputs, weight):
    n, ch, h, w = inputs.shape
    x3 = inputs.reshape(n, ch, h * w)
    rn = jnp.sum(x3 * x3, axis=1).reshape(1, ROWS)
    wn = jnp.sum(weight * weight, axis=1)[:, None]
    idx, md = _assign_codes(x3, rn, weight, wn)
    idx2d = idx.reshape(ROWS // _IDX_CH, _IDX_CH)
    wpad = jnp.pad(weight, ((0, 0), (0, _GDIM - DIM)))
    quantized = _sc_gather(wpad, idx2d)[:, :DIM]
    m = jnp.sum(md) / (ROWS * DIM)
    c_loss = m + COMMITMENT_COST * m
    quantized = quantized.reshape(n, h, w, ch)
    quantized = jnp.transpose(quantized, (0, 3, 1, 2))
    return (c_loss, quantized)


# speedup vs baseline: 1.0104x; 1.0104x over previous
import functools

import jax
import jax.numpy as jnp
from jax import lax
from jax.experimental import pallas as pl
from jax.experimental.pallas import tpu as pltpu
from jax.experimental.pallas import tpu_sc as plsc

NUM_EMB = 8192
DIM = 64
ROWS = 16384
COMMITMENT_COST = 0.25

BLK_R = 512
BLK_K = 512
N_R = ROWS // BLK_R
N_KC = NUM_EMB // BLK_K
HW = 1024
R_PER_IMG = HW // BLK_R


def _dist_body(x_ref, rn_ref, w_ref, wn_ref, idx_ref, md_ref):
    row = pl.program_id(1)
    x_t = x_ref[...].reshape(DIM, BLK_R)
    rn = rn_ref[...]
    fiota0 = lax.broadcasted_iota(
        jnp.int32, (BLK_K, BLK_R), 0).astype(jnp.float32)
    s = None
    idx = None
    for c in range(N_KC):
        w = w_ref[pl.ds(c * BLK_K, BLK_K), :]
        wn = wn_ref[pl.ds(c * BLK_K, BLK_K), :]
        mm = lax.dot_general(w, x_t, (((1,), (0,)), ((), ())),
                             preferred_element_type=jnp.float32)
        d2 = (rn + wn) - 2.0 * mm
        dist = jnp.sqrt(jnp.maximum(d2, 0.0))
        s_c = jnp.min(dist, axis=0, keepdims=True)
        i_c = jnp.min(jnp.where(dist == s_c, fiota0, 65536.0), axis=0,
                      keepdims=True) + jnp.float32(c * BLK_K)
        if c == 0:
            s, idx = s_c, i_c
        else:
            idx = jnp.where(s_c < s, i_c, idx)
            s = jnp.minimum(s, s_c)
    idx_ref[pl.ds(row, 1), :] = idx.astype(jnp.int32)
    md_ref[pl.ds(row, 1), :] = s * s


def _assign_codes(x3, rn, weight, wn, interpret=False):
    return pl.pallas_call(
        _dist_body,
        grid=(N_R // 8, 8),
        in_specs=[
            pl.BlockSpec((1, DIM, BLK_R),
                         lambda i, j: ((8 * i + j) // R_PER_IMG, 0,
                                       (8 * i + j) % R_PER_IMG)),
            pl.BlockSpec((1, BLK_R), lambda i, j: (0, 8 * i + j)),
            pl.BlockSpec((NUM_EMB, DIM), lambda i, j: (0, 0)),
            pl.BlockSpec((NUM_EMB, 1), lambda i, j: (0, 0)),
        ],
        out_specs=[
            pl.BlockSpec((8, BLK_R), lambda i, j: (i, 0)),
            pl.BlockSpec((8, BLK_R), lambda i, j: (i, 0)),
        ],
        out_shape=[
            jax.ShapeDtypeStruct((N_R, BLK_R), jnp.int32),
            jax.ShapeDtypeStruct((N_R, BLK_R), jnp.float32),
        ],
        compiler_params=pltpu.CompilerParams(
            dimension_semantics=("parallel", "arbitrary")),
        interpret=interpret,
    )(x3, rn, weight, wn)


_NW = 32
_B_PER_W = ROWS // _NW
_IDX_CH = 128
_CH_PER_W = _B_PER_W // _IDX_CH
_GDIM = 128


def _make_sc_gather():
    mesh = plsc.VectorSubcoreMesh(core_axis_name="c", subcore_axis_name="s")

    @functools.partial(
        pl.kernel,
        mesh=mesh,
        out_type=jax.ShapeDtypeStruct((ROWS, _GDIM), jnp.float32),
        scratch_types=[
            pltpu.VMEM((_CH_PER_W, _IDX_CH), jnp.int32),
            pltpu.VMEM((_B_PER_W, _GDIM), jnp.float32),
            pltpu.SemaphoreType.DMA,
        ],
    )
    def gather_k(table_hbm, idx_hbm, out_hbm, idx_v, rows_v, sem):
        wid = lax.axis_index("s") * 2 + lax.axis_index("c")
        base = wid * _B_PER_W
        pltpu.sync_copy(idx_hbm.at[pl.ds(wid * _CH_PER_W, _CH_PER_W)], idx_v)
        copies = []
        for j in range(_CH_PER_W):
            copies.append(pltpu.async_copy(
                table_hbm.at[idx_v.at[j]],
                rows_v.at[pl.ds(j * _IDX_CH, _IDX_CH)], sem))
        for cp in copies:
            cp.wait()
        pltpu.sync_copy(rows_v, out_hbm.at[pl.ds(base, _B_PER_W)])

    return gather_k


_sc_gather_cache = []


def _sc_gather(table, idx2d):
    if not _sc_gather_cache:
        _sc_gather_cache.append(_make_sc_gather())
    return _sc_gather_cache[0](table, idx2d)


def kernel(inputs, weight):
    n, ch, h, w = inputs.shape
    x3 = inputs.reshape(n, ch, h * w)
    rn = jnp.sum(x3 * x3, axis=1).reshape(1, ROWS)
    wn = jnp.sum(weight * weight, axis=1)[:, None]
    idx, md = _assign_codes(x3, rn, weight, wn)
    idx2d = idx.reshape(ROWS // _IDX_CH, _IDX_CH)
    wpad = jnp.pad(weight, ((0, 0), (0, _GDIM - DIM)))
    quantized = _sc_gather(wpad, idx2d)[:, :DIM]
    m = jnp.sum(md) / (ROWS * DIM)
    c_loss = m + COMMITMENT_COST * m
    quantized = quantized.reshape(n, h, w, ch)
    quantized = jnp.transpose(quantized, (0, 3, 1, 2))
    return (c_loss, quantized)
